# baseline (speedup 1.0000x reference)
import functools

import jax
import jax.numpy as jnp
from jax import lax
from jax.experimental import pallas as pl
from jax.experimental.pallas import tpu as pltpu
from jax.experimental.pallas import tpu_sc as plsc

_NC = 2
_NS = 16
_L = 16
_NW = _NC * _NS


@functools.cache
def _sc_lookup_call(n: int, conv_words: int):
    n_per_w = n // _NW
    mesh = plsc.VectorSubcoreMesh(core_axis_name="c", subcore_axis_name="s")

    @functools.partial(
        pl.kernel,
        out_type=jax.ShapeDtypeStruct((n,), jnp.int32),
        mesh=mesh,
        scratch_types=[
            pltpu.VMEM((conv_words,), jnp.int32),
            pltpu.VMEM((n_per_w,), jnp.int32),
            pltpu.VMEM((n_per_w,), jnp.int32),
        ],
        compiler_params=pltpu.CompilerParams(needs_layout_passes=False),
    )
    def body(sp_hbm, conv_hbm, out_hbm, conv_v, sp_v, out_v):
        wid = lax.axis_index("s") * jnp.int32(_NC) + lax.axis_index("c")
        base = wid * jnp.int32(n_per_w)
        pltpu.sync_copy(conv_hbm, conv_v)
        pltpu.sync_copy(sp_hbm.at[pl.ds(base, n_per_w)], sp_v)

        @plsc.parallel_loop(jnp.int32(0), jnp.int32(n_per_w),
                            step=jnp.int32(_L), unroll=8)
        def _(off):
            idx = sp_v[pl.ds(off, _L)]
            out_v[pl.ds(off, _L)] = plsc.load_gather(conv_v, [idx])

        pltpu.sync_copy(out_v, out_hbm.at[pl.ds(base, n_per_w)])

    return body


def kernel(species, coordinates, conv_tensor):
    shape = species.shape
    n = species.size
    conv16 = (
        jnp.zeros((_L,), jnp.int32)
        .at[: conv_tensor.shape[0]]
        .set(conv_tensor.astype(jnp.int32))
    )
    sp32 = species.reshape(n).astype(jnp.int32)
    out32 = _sc_lookup_call(n, _L)(sp32, conv16)
    return out32.reshape(shape).astype(conv_tensor.dtype), coordinates
